# Initial kernel scaffold; baseline (speedup 1.0000x reference)
import jax
import jax.numpy as jnp
from jax.experimental import pallas as pl


def kernel(x, edge_index, W):
    raise NotImplementedError("write your pallas kernel here")


# speedup vs baseline: 6.8695x; 6.8695x over previous
import functools

import jax
import jax.numpy as jnp
from jax import lax
from jax.experimental import pallas as pl
from jax.experimental.pallas import tpu as pltpu
from jax.experimental.pallas import tpu_sc as plsc

L = 2048
N = 32
A = 21
E = 16384
NW = 32
EPW = E // NW
BK = 64
NB = EPW // BK
ROW = A * A + 1
RPW = L // NW
X1W = RPW * A

_mesh = plsc.VectorSubcoreMesh(core_axis_name="c", subcore_axis_name="s")


@functools.partial(
    pl.kernel,
    mesh=_mesh,
    out_type=[
        jax.ShapeDtypeStruct((L * A,), jnp.float32),
        jax.ShapeDtypeStruct((E * ROW,), jnp.float32),
    ],
    scratch_types=[
        pltpu.VMEM((L * N,), jnp.int32),
        pltpu.VMEM((EPW,), jnp.int32),
        pltpu.VMEM((EPW,), jnp.int32),
        pltpu.VMEM((N,), jnp.float32),
        pltpu.VMEM((N,), jnp.float32),
        pltpu.VMEM((N,), jnp.float32),
        pltpu.VMEM((BK * ROW + 16,), jnp.float32),
        pltpu.VMEM((BK * ROW + 16,), jnp.float32),
        pltpu.VMEM((X1W,), jnp.float32),
        pltpu.SemaphoreType.DMA,
        pltpu.SemaphoreType.DMA,
        pltpu.SemaphoreType.DMA,
    ],
    compiler_params=pltpu.CompilerParams(
        needs_layout_passes=False, use_tc_tiling_on_sc=False
    ),
)
def _msa_sc(x_hbm, e0_hbm, e1_hbm, w_hbm, x1_hbm, x2_hbm,
            x_v, e0_v, e1_v, w_v, hi, hj, stage_a, stage_b, x1_st,
            sem_a, sem_b, sem_x1):
    c = lax.axis_index("c")
    s = lax.axis_index("s")
    w = s * 2 + c

    pltpu.sync_copy(x_hbm, x_v)
    pltpu.sync_copy(e0_hbm.at[pl.ds(w * EPW, EPW)], e0_v)
    pltpu.sync_copy(e1_hbm.at[pl.ds(w * EPW, EPW)], e1_v)
    pltpu.sync_copy(w_hbm, w_v)

    iota = lax.iota(jnp.int32, 16)
    zero = jnp.zeros((16,), jnp.float32)

    w0 = w_v[pl.ds(0, 16)]
    w1 = w_v[pl.ds(16, 16)]
    m = jnp.maximum(jnp.max(w0), jnp.max(w1))
    ew0 = jnp.exp(w0 - m)
    ew1 = jnp.exp(w1 - m)
    wsum = jnp.sum(ew0) + jnp.sum(ew1)
    wsm0 = ew0 / wsum
    wsm1 = ew1 / wsum
    wng0 = -wsm0
    wng1 = -wsm1

    tail_mask = jnp.where(iota < 9, 1.0, 0.0).astype(jnp.float32)

    def x1_body(r, carry):
        g = w * RPW + r
        hi[pl.ds(0, 16)] = zero
        hi[pl.ds(16, 16)] = zero
        xr0 = plsc.load_gather(x_v, [g * N + iota])
        xr1 = plsc.load_gather(x_v, [g * N + 16 + iota])
        plsc.addupdate_scatter(hi, [xr0], wsm0)
        plsc.addupdate_scatter(hi, [xr1], wsm1)
        x1_st[pl.ds(r * A, 16)] = hi[pl.ds(0, 16)]
        x1_st[pl.ds(r * A + 5, 16)] = hi[pl.ds(5, 16)]
        return carry

    lax.fori_loop(0, RPW, x1_body, 0)
    cp_x1 = pltpu.async_copy(
        x1_st, x1_hbm.at[pl.ds(w * X1W, X1W)], sem_x1
    )

    def edge_body_for(stage):
        def edge_body(k, bb):
            kk = bb + k
            kv = jnp.full((16,), kk, jnp.int32)
            iv = plsc.load_gather(e0_v, [kv])
            jv = plsc.load_gather(e1_v, [kv])
            xi0 = plsc.load_gather(x_v, [iv * N + iota])
            xi1 = plsc.load_gather(x_v, [iv * N + 16 + iota])
            xj0 = plsc.load_gather(x_v, [jv * N + iota])
            xj1 = plsc.load_gather(x_v, [jv * N + 16 + iota])

            hi[pl.ds(0, 16)] = zero
            hi[pl.ds(16, 16)] = zero
            hj[pl.ds(0, 16)] = zero
            hj[pl.ds(16, 16)] = zero
            plsc.addupdate_scatter(hi, [xi0], wsm0)
            plsc.addupdate_scatter(hi, [xi1], wsm1)
            plsc.addupdate_scatter(hj, [xj0], wng0)
            plsc.addupdate_scatter(hj, [xj1], wng1)

            hj_lo = hj[pl.ds(0, 16)]
            hj_sh = hj[pl.ds(5, 16)]
            hi_v0 = hi[pl.ds(0, 16)]
            hi_v1 = hi[pl.ds(16, 16)]

            base = k * ROW
            for a in range(A):
                sa_s = hi_v0[a] if a < 16 else hi_v1[a - 16]
                sa = jnp.full((16,), sa_s, jnp.float32)
                stage[pl.ds(base + a * A, 16)] = sa * hj_lo
                stage[pl.ds(base + a * A + 5, 16)] = sa * hj_sh

            p0 = base + xi0 * A + xj0
            p1 = base + xi1 * A + xj1
            plsc.addupdate_scatter(stage, [p0], wsm0)
            plsc.addupdate_scatter(stage, [p1], wsm1)

            acc = zero
            for t in range(27):
                v = stage[pl.ds(base + 16 * t, 16)]
                acc = acc + v * v
            v27 = stage[pl.ds(base + 432, 16)] * tail_mask
            acc = acc + v27 * v27
            ssq = jnp.sum(acc) + jnp.float32(1e-12)

            xv = jnp.full((16,), ssq, jnp.float32)
            bi = lax.bitcast_convert_type(xv, jnp.int32)
            r = lax.bitcast_convert_type(
                jnp.int32(0x5F3759DF) - lax.shift_right_logical(bi, 1),
                jnp.float32,
            )
            half = jnp.float32(0.5) * xv
            for _ in range(3):
                r = r * (jnp.float32(1.5) - half * r * r)
            normv = xv * r
            v27f = jnp.where(iota == 9, normv, v27)
            stage[pl.ds(base + 432, 16)] = v27f
            return bb

        return edge_body

    stages = (stage_a, stage_b)
    sems = (sem_a, sem_b)
    pending = [None, None]
    for b in range(NB):
        sl = b % 2
        if pending[sl] is not None:
            pending[sl].wait()
        st = stages[sl]
        lax.fori_loop(0, BK, edge_body_for(st), b * BK)
        off = (w * EPW + b * BK) * ROW
        pending[sl] = pltpu.async_copy(
            st.at[pl.ds(0, BK * ROW)], x2_hbm.at[pl.ds(off, BK * ROW)],
            sems[sl],
        )
    pending[0].wait()
    pending[1].wait()
    cp_x1.wait()


def kernel(x, edge_index, W):
    xf = x[:, :N].astype(jnp.int32).reshape(-1)
    e0 = edge_index[0].astype(jnp.int32)
    e1 = edge_index[1].astype(jnp.int32)
    wf = W.astype(jnp.float32).reshape(N)
    x1f, x2f = _msa_sc(xf, e0, e1, wf)
    return x1f.reshape(L, A), x2f.reshape(E, ROW)


# speedup vs baseline: 7.4012x; 1.0774x over previous
import functools

import jax
import jax.numpy as jnp
from jax import lax
from jax.experimental import pallas as pl
from jax.experimental.pallas import tpu as pltpu
from jax.experimental.pallas import tpu_sc as plsc

L = 2048
N = 32
A = 21
E = 16384
NW = 32
EPW = E // NW
BK = 64
NB = EPW // BK
ROW = A * A + 1
RPW = L // NW
X1W = RPW * A

_mesh = plsc.VectorSubcoreMesh(core_axis_name="c", subcore_axis_name="s")


@functools.partial(
    pl.kernel,
    mesh=_mesh,
    out_type=[
        jax.ShapeDtypeStruct((L * A,), jnp.float32),
        jax.ShapeDtypeStruct((E * ROW,), jnp.float32),
    ],
    scratch_types=[
        pltpu.VMEM((L * N,), jnp.int32),
        pltpu.VMEM((EPW,), jnp.int32),
        pltpu.VMEM((EPW,), jnp.int32),
        pltpu.VMEM((N,), jnp.float32),
        pltpu.VMEM((N,), jnp.float32),
        pltpu.VMEM((N,), jnp.float32),
        pltpu.VMEM((BK * ROW + 16,), jnp.float32),
        pltpu.VMEM((BK * ROW + 16,), jnp.float32),
        pltpu.VMEM((X1W,), jnp.float32),
        pltpu.SemaphoreType.DMA,
        pltpu.SemaphoreType.DMA,
        pltpu.SemaphoreType.DMA,
    ],
    compiler_params=pltpu.CompilerParams(
        needs_layout_passes=False, use_tc_tiling_on_sc=False
    ),
)
def _msa_sc(x_hbm, e0_hbm, e1_hbm, w_hbm, x1_hbm, x2_hbm,
            x_v, e0_v, e1_v, w_v, hi, hj, stage_a, stage_b, x1_st,
            sem_a, sem_b, sem_x1):
    c = lax.axis_index("c")
    s = lax.axis_index("s")
    w = s * 2 + c

    pltpu.sync_copy(x_hbm, x_v)
    pltpu.sync_copy(e0_hbm.at[pl.ds(w * EPW, EPW)], e0_v)
    pltpu.sync_copy(e1_hbm.at[pl.ds(w * EPW, EPW)], e1_v)
    pltpu.sync_copy(w_hbm, w_v)

    iota = lax.iota(jnp.int32, 16)
    zero = jnp.zeros((16,), jnp.float32)

    w0 = w_v[pl.ds(0, 16)]
    w1 = w_v[pl.ds(16, 16)]
    m = jnp.maximum(jnp.max(w0), jnp.max(w1))
    ew0 = jnp.exp(w0 - m)
    ew1 = jnp.exp(w1 - m)
    wsum = jnp.sum(ew0) + jnp.sum(ew1)
    wsm0 = ew0 / wsum
    wsm1 = ew1 / wsum
    wng0 = -wsm0
    wng1 = -wsm1

    tail_mask = jnp.where(iota < 9, 1.0, 0.0).astype(jnp.float32)

    def x1_body(r, carry):
        g = w * RPW + r
        hi[pl.ds(0, 16)] = zero
        hi[pl.ds(16, 16)] = zero
        xr0 = plsc.load_gather(x_v, [g * N + iota])
        xr1 = plsc.load_gather(x_v, [g * N + 16 + iota])
        plsc.addupdate_scatter(hi, [xr0], wsm0)
        plsc.addupdate_scatter(hi, [xr1], wsm1)
        x1_st[pl.ds(r * A, 16)] = hi[pl.ds(0, 16)]
        x1_st[pl.ds(r * A + 5, 16)] = hi[pl.ds(5, 16)]
        return carry

    lax.fori_loop(0, RPW, x1_body, 0)
    cp_x1 = pltpu.async_copy(
        x1_st, x1_hbm.at[pl.ds(w * X1W, X1W)], sem_x1
    )

    def edge_body_for(stage):
        def edge_body(k, bb):
            kk = bb + k
            kv = jnp.full((16,), kk, jnp.int32)
            iv = plsc.load_gather(e0_v, [kv])
            jv = plsc.load_gather(e1_v, [kv])
            xi0 = plsc.load_gather(x_v, [iv * N + iota])
            xi1 = plsc.load_gather(x_v, [iv * N + 16 + iota])
            xj0 = plsc.load_gather(x_v, [jv * N + iota])
            xj1 = plsc.load_gather(x_v, [jv * N + 16 + iota])

            hi[pl.ds(0, 16)] = zero
            hi[pl.ds(16, 16)] = zero
            hj[pl.ds(0, 16)] = zero
            hj[pl.ds(16, 16)] = zero
            plsc.addupdate_scatter(hi, [xi0], wsm0)
            plsc.addupdate_scatter(hi, [xi1], wsm1)
            plsc.addupdate_scatter(hj, [xj0], wng0)
            plsc.addupdate_scatter(hj, [xj1], wng1)

            hj_lo = hj[pl.ds(0, 16)]
            hj_sh = hj[pl.ds(5, 16)]
            hi_v0 = hi[pl.ds(0, 16)]
            hi_v1 = hi[pl.ds(16, 16)]

            base = k * ROW
            for a in range(A):
                sa_s = hi_v0[a] if a < 16 else hi_v1[a - 16]
                sa = jnp.full((16,), sa_s, jnp.float32)
                stage[pl.ds(base + a * A, 16)] = sa * hj_lo
                stage[pl.ds(base + a * A + 5, 16)] = sa * hj_sh

            p0 = base + xi0 * A + xj0
            p1 = base + xi1 * A + xj1
            old0 = plsc.load_gather(stage, [p0])
            old1 = plsc.load_gather(stage, [p1])
            plsc.addupdate_scatter(stage, [p0], wsm0)
            plsc.addupdate_scatter(stage, [p1], wsm1)
            new0 = plsc.load_gather(stage, [p0])
            new1 = plsc.load_gather(stage, [p1])

            si = jnp.sum(hi_v0 * hi_v0 + hi_v1 * hi_v1)
            hj_v1 = hj[pl.ds(16, 16)]
            sj = jnp.sum(hj_lo * hj_lo + hj_v1 * hj_v1)
            cross = jnp.sum(wsm0 * (old0 + new0) + wsm1 * (old1 + new1))
            ssq = si * sj + cross + jnp.float32(1e-12)

            xv = jnp.full((16,), ssq, jnp.float32)
            bi = lax.bitcast_convert_type(xv, jnp.int32)
            r = lax.bitcast_convert_type(
                jnp.int32(0x5F3759DF) - lax.shift_right_logical(bi, 1),
                jnp.float32,
            )
            half = jnp.float32(0.5) * xv
            for _ in range(3):
                r = r * (jnp.float32(1.5) - half * r * r)
            normv = xv * r
            v27 = stage[pl.ds(base + 432, 16)]
            v27f = jnp.where(iota == 9, normv, v27)
            stage[pl.ds(base + 432, 16)] = v27f
            return bb

        return edge_body

    stages = (stage_a, stage_b)
    sems = (sem_a, sem_b)
    pending = [None, None]
    for b in range(NB):
        sl = b % 2
        if pending[sl] is not None:
            pending[sl].wait()
        st = stages[sl]
        lax.fori_loop(0, BK, edge_body_for(st), b * BK, unroll=2)
        off = (w * EPW + b * BK) * ROW
        pending[sl] = pltpu.async_copy(
            st.at[pl.ds(0, BK * ROW)], x2_hbm.at[pl.ds(off, BK * ROW)],
            sems[sl],
        )
    pending[0].wait()
    pending[1].wait()
    cp_x1.wait()


def kernel(x, edge_index, W):
    xf = x[:, :N].astype(jnp.int32).reshape(-1)
    e0 = edge_index[0].astype(jnp.int32)
    e1 = edge_index[1].astype(jnp.int32)
    wf = W.astype(jnp.float32).reshape(N)
    x1f, x2f = _msa_sc(xf, e0, e1, wf)
    return x1f.reshape(L, A), x2f.reshape(E, ROW)


# speedup vs baseline: 7.6064x; 1.0277x over previous
import functools

import jax
import jax.numpy as jnp
from jax import lax
from jax.experimental import pallas as pl
from jax.experimental.pallas import tpu as pltpu
from jax.experimental.pallas import tpu_sc as plsc

L = 2048
N = 32
A = 21
E = 16384
NW = 32
EPW = E // NW
BK = 64
NB = EPW // BK
ROW = A * A + 1
RPW = L // NW
X1W = RPW * A

_mesh = plsc.VectorSubcoreMesh(core_axis_name="c", subcore_axis_name="s")


@functools.partial(
    pl.kernel,
    mesh=_mesh,
    out_type=[
        jax.ShapeDtypeStruct((L * A,), jnp.float32),
        jax.ShapeDtypeStruct((E * ROW,), jnp.float32),
    ],
    scratch_types=[
        pltpu.VMEM_SHARED((L, N), jnp.int32),
        pltpu.VMEM((L, N), jnp.int32),
        pltpu.VMEM((EPW,), jnp.int32),
        pltpu.VMEM((EPW,), jnp.int32),
        pltpu.VMEM((N,), jnp.float32),
        pltpu.VMEM((N,), jnp.float32),
        pltpu.VMEM((N,), jnp.float32),
        pltpu.VMEM((BK * ROW + 16,), jnp.float32),
        pltpu.VMEM((BK * ROW + 16,), jnp.float32),
        pltpu.VMEM((X1W,), jnp.float32),
        pltpu.SemaphoreType.DMA,
        pltpu.SemaphoreType.DMA,
        pltpu.SemaphoreType.DMA,
    ],
    compiler_params=pltpu.CompilerParams(
        needs_layout_passes=False, use_tc_tiling_on_sc=False
    ),
)
def _msa_sc(x_hbm, e0_hbm, e1_hbm, w_hbm, x1_hbm, x2_hbm,
            x_sh, x_v, e0_v, e1_v, w_v, hi, hj, stage_a, stage_b, x1_st,
            sem_a, sem_b, sem_x1):
    c = lax.axis_index("c")
    s = lax.axis_index("s")
    w = s * 2 + c

    @pl.when(s == 0)
    def _():
        pltpu.sync_copy(x_hbm, x_sh)

    pltpu.sync_copy(e0_hbm.at[pl.ds(w * EPW, EPW)], e0_v)
    pltpu.sync_copy(e1_hbm.at[pl.ds(w * EPW, EPW)], e1_v)
    pltpu.sync_copy(w_hbm, w_v)
    plsc.subcore_barrier()
    pltpu.sync_copy(x_sh, x_v)

    iota = lax.iota(jnp.int32, 16)
    zero = jnp.zeros((16,), jnp.float32)

    w0 = w_v[pl.ds(0, 16)]
    w1 = w_v[pl.ds(16, 16)]
    m = jnp.maximum(jnp.max(w0), jnp.max(w1))
    ew0 = jnp.exp(w0 - m)
    ew1 = jnp.exp(w1 - m)
    wsum = jnp.sum(ew0) + jnp.sum(ew1)
    wsm0 = ew0 / wsum
    wsm1 = ew1 / wsum
    wng0 = -wsm0
    wng1 = -wsm1

    tail_mask = jnp.where(iota < 9, 1.0, 0.0).astype(jnp.float32)

    def x1_body(r, carry):
        g = w * RPW + r
        hi[pl.ds(0, 16)] = zero
        hi[pl.ds(16, 16)] = zero
        gv = jnp.full((16,), g, jnp.int32)
        xr0 = plsc.load_gather(x_v, [gv, iota])
        xr1 = plsc.load_gather(x_v, [gv, 16 + iota])
        plsc.addupdate_scatter(hi, [xr0], wsm0)
        plsc.addupdate_scatter(hi, [xr1], wsm1)
        x1_st[pl.ds(r * A, 16)] = hi[pl.ds(0, 16)]
        x1_st[pl.ds(r * A + 5, 16)] = hi[pl.ds(5, 16)]
        return carry

    lax.fori_loop(0, RPW, x1_body, 0)
    cp_x1 = pltpu.async_copy(
        x1_st, x1_hbm.at[pl.ds(w * X1W, X1W)], sem_x1
    )

    def edge_body_for(stage):
        def edge_body(k, bb):
            kk = bb + k
            kv = jnp.full((16,), kk, jnp.int32)
            iv = plsc.load_gather(e0_v, [kv])
            jv = plsc.load_gather(e1_v, [kv])
            xi0 = plsc.load_gather(x_v, [iv, iota])
            xi1 = plsc.load_gather(x_v, [iv, 16 + iota])
            xj0 = plsc.load_gather(x_v, [jv, iota])
            xj1 = plsc.load_gather(x_v, [jv, 16 + iota])

            hi[pl.ds(0, 16)] = zero
            hi[pl.ds(16, 16)] = zero
            hj[pl.ds(0, 16)] = zero
            hj[pl.ds(16, 16)] = zero
            plsc.addupdate_scatter(hi, [xi0], wsm0)
            plsc.addupdate_scatter(hi, [xi1], wsm1)
            plsc.addupdate_scatter(hj, [xj0], wng0)
            plsc.addupdate_scatter(hj, [xj1], wng1)

            hj_lo = hj[pl.ds(0, 16)]
            hj_sh = hj[pl.ds(5, 16)]
            hi_v0 = hi[pl.ds(0, 16)]
            hi_v1 = hi[pl.ds(16, 16)]

            base = k * ROW
            for a in range(A):
                sa_s = hi_v0[a] if a < 16 else hi_v1[a - 16]
                sa = jnp.full((16,), sa_s, jnp.float32)
                stage[pl.ds(base + a * A, 16)] = sa * hj_lo
                stage[pl.ds(base + a * A + 5, 16)] = sa * hj_sh

            p0 = base + xi0 * A + xj0
            p1 = base + xi1 * A + xj1
            old0 = plsc.load_gather(stage, [p0])
            old1 = plsc.load_gather(stage, [p1])
            plsc.addupdate_scatter(stage, [p0], wsm0)
            plsc.addupdate_scatter(stage, [p1], wsm1)
            new0 = plsc.load_gather(stage, [p0])
            new1 = plsc.load_gather(stage, [p1])

            si = jnp.sum(hi_v0 * hi_v0 + hi_v1 * hi_v1)
            hj_v1 = hj[pl.ds(16, 16)]
            sj = jnp.sum(hj_lo * hj_lo + hj_v1 * hj_v1)
            cross = jnp.sum(wsm0 * (old0 + new0) + wsm1 * (old1 + new1))
            ssq = si * sj + cross + jnp.float32(1e-12)

            xv = jnp.full((16,), ssq, jnp.float32)
            bi = lax.bitcast_convert_type(xv, jnp.int32)
            r = lax.bitcast_convert_type(
                jnp.int32(0x5F3759DF) - lax.shift_right_logical(bi, 1),
                jnp.float32,
            )
            half = jnp.float32(0.5) * xv
            for _ in range(3):
                r = r * (jnp.float32(1.5) - half * r * r)
            normv = xv * r
            v27 = stage[pl.ds(base + 432, 16)]
            v27f = jnp.where(iota == 9, normv, v27)
            stage[pl.ds(base + 432, 16)] = v27f
            return bb

        return edge_body

    stages = (stage_a, stage_b)
    sems = (sem_a, sem_b)
    pending = [None, None]
    for b in range(NB):
        sl = b % 2
        if pending[sl] is not None:
            pending[sl].wait()
        st = stages[sl]
        lax.fori_loop(0, BK, edge_body_for(st), b * BK, unroll=2)
        off = (w * EPW + b * BK) * ROW
        pending[sl] = pltpu.async_copy(
            st.at[pl.ds(0, BK * ROW)], x2_hbm.at[pl.ds(off, BK * ROW)],
            sems[sl],
        )
    pending[0].wait()
    pending[1].wait()
    cp_x1.wait()


def kernel(x, edge_index, W):
    xf = x[:, :N].astype(jnp.int32)
    e0 = edge_index[0].astype(jnp.int32)
    e1 = edge_index[1].astype(jnp.int32)
    wf = W.astype(jnp.float32).reshape(N)
    x1f, x2f = _msa_sc(xf, e0, e1, wf)
    return x1f.reshape(L, A), x2f.reshape(E, ROW)


# speedup vs baseline: 8.0323x; 1.0560x over previous
import functools

import jax
import jax.numpy as jnp
from jax import lax
from jax.experimental import pallas as pl
from jax.experimental.pallas import tpu as pltpu
from jax.experimental.pallas import tpu_sc as plsc

L = 2048
N = 32
A = 21
E = 16384
NW = 32
EPW = E // NW
BK = 64
NB = EPW // BK
ROW = A * A + 1
RPW = L // NW
X1W = RPW * A

_mesh = plsc.VectorSubcoreMesh(core_axis_name="c", subcore_axis_name="s")


@functools.partial(
    pl.kernel,
    mesh=_mesh,
    out_type=[
        jax.ShapeDtypeStruct((L * A,), jnp.float32),
        jax.ShapeDtypeStruct((E, ROW), jnp.float32),
    ],
    scratch_types=[
        pltpu.VMEM((EPW,), jnp.int32),
        pltpu.VMEM((EPW,), jnp.int32),
        pltpu.VMEM((N,), jnp.float32),
        pltpu.VMEM((N,), jnp.float32),
        pltpu.VMEM((N,), jnp.float32),
        pltpu.VMEM((RPW, N), jnp.int32),
        pltpu.VMEM((BK, N), jnp.int32),
        pltpu.VMEM((BK, N), jnp.int32),
        pltpu.VMEM((BK, N), jnp.int32),
        pltpu.VMEM((BK, N), jnp.int32),
        pltpu.VMEM((BK, ROW), jnp.float32),
        pltpu.VMEM((BK, ROW), jnp.float32),
        pltpu.VMEM((X1W,), jnp.float32),
        pltpu.SemaphoreType.DMA,
        pltpu.SemaphoreType.DMA,
        pltpu.SemaphoreType.DMA,
        pltpu.SemaphoreType.DMA,
        pltpu.SemaphoreType.DMA,
    ],
    compiler_params=pltpu.CompilerParams(
        needs_layout_passes=False, use_tc_tiling_on_sc=False
    ),
)
def _msa_sc(x_hbm, e0_hbm, e1_hbm, w_hbm, x1_hbm, x2_hbm,
            e0_v, e1_v, w_v, hi, hj, xrow_v,
            ri_a, ri_b, rj_a, rj_b, stage_a, stage_b, x1_st,
            sem_a, sem_b, sem_ra, sem_rb, sem_x1):
    c = lax.axis_index("c")
    s = lax.axis_index("s")
    w = s * 2 + c

    pltpu.sync_copy(e0_hbm.at[pl.ds(w * EPW, EPW)], e0_v)
    pltpu.sync_copy(e1_hbm.at[pl.ds(w * EPW, EPW)], e1_v)
    pltpu.sync_copy(w_hbm, w_v)

    rows = (ri_a, ri_b, rj_a, rj_b)
    rsems = (sem_ra, sem_rb)

    def prefetch(b):
        sl = b % 2
        cpi = pltpu.async_copy(
            x_hbm.at[e0_v.at[pl.ds(b * BK, BK)]], rows[sl], rsems[sl]
        )
        cpj = pltpu.async_copy(
            x_hbm.at[e1_v.at[pl.ds(b * BK, BK)]], rows[2 + sl], rsems[sl]
        )
        return cpi, cpj

    pend_rows = prefetch(0)

    pltpu.sync_copy(x_hbm.at[pl.ds(w * RPW, RPW)], xrow_v)

    iota = lax.iota(jnp.int32, 16)
    zero = jnp.zeros((16,), jnp.float32)

    w0 = w_v[pl.ds(0, 16)]
    w1 = w_v[pl.ds(16, 16)]
    m = jnp.maximum(jnp.max(w0), jnp.max(w1))
    ew0 = jnp.exp(w0 - m)
    ew1 = jnp.exp(w1 - m)
    wsum = jnp.sum(ew0) + jnp.sum(ew1)
    wsm0 = ew0 / wsum
    wsm1 = ew1 / wsum
    wng0 = -wsm0
    wng1 = -wsm1

    def x1_body(r, carry):
        hi[pl.ds(0, 16)] = zero
        hi[pl.ds(16, 16)] = zero
        plsc.addupdate_scatter(hi, [xrow_v[r, pl.ds(0, 16)]], wsm0)
        plsc.addupdate_scatter(hi, [xrow_v[r, pl.ds(16, 16)]], wsm1)
        x1_st[pl.ds(r * A, 16)] = hi[pl.ds(0, 16)]
        x1_st[pl.ds(r * A + 5, 16)] = hi[pl.ds(5, 16)]
        return carry

    lax.fori_loop(0, RPW, x1_body, 0)
    cp_x1 = pltpu.async_copy(
        x1_st, x1_hbm.at[pl.ds(w * X1W, X1W)], sem_x1
    )

    def edge_body_for(stage, ri, rj):
        def edge_body(k, carry):
            xi0 = ri[k, pl.ds(0, 16)]
            xi1 = ri[k, pl.ds(16, 16)]
            xj0 = rj[k, pl.ds(0, 16)]
            xj1 = rj[k, pl.ds(16, 16)]

            hi[pl.ds(0, 16)] = zero
            hi[pl.ds(16, 16)] = zero
            hj[pl.ds(0, 16)] = zero
            hj[pl.ds(16, 16)] = zero
            plsc.addupdate_scatter(hi, [xi0], wsm0)
            plsc.addupdate_scatter(hi, [xi1], wsm1)
            plsc.addupdate_scatter(hj, [xj0], wng0)
            plsc.addupdate_scatter(hj, [xj1], wng1)

            hj_lo = hj[pl.ds(0, 16)]
            hj_sh = hj[pl.ds(5, 16)]
            hi_v0 = hi[pl.ds(0, 16)]
            hi_v1 = hi[pl.ds(16, 16)]

            for a in range(A):
                sa_s = hi_v0[a] if a < 16 else hi_v1[a - 16]
                sa = jnp.full((16,), sa_s, jnp.float32)
                stage[k, pl.ds(a * A, 16)] = sa * hj_lo
                stage[k, pl.ds(a * A + 5, 16)] = sa * hj_sh

            kv = jnp.full((16,), k, jnp.int32)
            p0 = xi0 * A + xj0
            p1 = xi1 * A + xj1
            old0 = plsc.load_gather(stage, [kv, p0])
            old1 = plsc.load_gather(stage, [kv, p1])
            plsc.addupdate_scatter(stage, [kv, p0], wsm0)
            plsc.addupdate_scatter(stage, [kv, p1], wsm1)
            new0 = plsc.load_gather(stage, [kv, p0])
            new1 = plsc.load_gather(stage, [kv, p1])

            si = jnp.sum(hi_v0 * hi_v0 + hi_v1 * hi_v1)
            hj_v1 = hj[pl.ds(16, 16)]
            sj = jnp.sum(hj_lo * hj_lo + hj_v1 * hj_v1)
            cross = jnp.sum(wsm0 * (old0 + new0) + wsm1 * (old1 + new1))
            ssq = si * sj + cross + jnp.float32(1e-12)

            xv = jnp.full((16,), ssq, jnp.float32)
            bi = lax.bitcast_convert_type(xv, jnp.int32)
            r = lax.bitcast_convert_type(
                jnp.int32(0x5F3759DF) - lax.shift_right_logical(bi, 1),
                jnp.float32,
            )
            half = jnp.float32(0.5) * xv
            for _ in range(3):
                r = r * (jnp.float32(1.5) - half * r * r)
            normv = xv * r
            v26 = stage[k, pl.ds(426, 16)]
            stage[k, pl.ds(426, 16)] = jnp.where(iota == 15, normv, v26)
            return carry

        return edge_body

    stages = (stage_a, stage_b)
    osems = (sem_a, sem_b)
    pending = [None, None]
    for b in range(NB):
        sl = b % 2
        if b + 1 < NB:
            nxt = prefetch(b + 1)
        else:
            nxt = None
        for cp in pend_rows:
            cp.wait()
        if pending[sl] is not None:
            pending[sl].wait()
        lax.fori_loop(
            0, BK,
            edge_body_for(stages[sl], rows[sl], rows[2 + sl]),
            0, unroll=2,
        )
        pending[sl] = pltpu.async_copy(
            stages[sl], x2_hbm.at[pl.ds(w * EPW + b * BK, BK)], osems[sl]
        )
        pend_rows = nxt
    pending[0].wait()
    pending[1].wait()
    cp_x1.wait()


def kernel(x, edge_index, W):
    xf = x[:, :N].astype(jnp.int32)
    e0 = edge_index[0].astype(jnp.int32)
    e1 = edge_index[1].astype(jnp.int32)
    wf = W.astype(jnp.float32).reshape(N)
    x1f, x2f = _msa_sc(xf, e0, e1, wf)
    return x1f.reshape(L, A), x2f
